# Initial kernel scaffold; baseline (speedup 1.0000x reference)
import jax
import jax.numpy as jnp
from jax.experimental import pallas as pl


def kernel(x, g_norm, Wq, Wk, Wv, k_pe, v_pe, Wck, Wcv, mem_kv, W_comb, b_comb, Wo):
    raise NotImplementedError("write your pallas kernel here")


# speedup vs baseline: 2.5692x; 2.5692x over previous
import jax
import jax.numpy as jnp
from jax.experimental import pallas as pl

B, N, D = 1, 2048, 768
H, KVH, DH = 12, 12, 64
BLK, STRIDE = 16, 8
SELBLK, NSEL = 16, 4
WIN = 64
SCALE = DH ** -0.5
NCB = (N - BLK) // STRIDE + 1
NSB = N // SELBLK
TQ = 256
BF16 = jnp.bfloat16
F32 = jnp.float32


def _bdot(a, b, dims=None):
    if dims is None:
        dims = (((a.ndim - 1,), (0,)), ((), ()))
    return jax.lax.dot_general(a.astype(BF16), b.astype(BF16), dims,
                               preferred_element_type=F32)


def _proj_kernel(x_ref, g_ref, wq_ref, wk_ref, wv_ref, wc_ref, bc_ref,
                 q_ref, k_ref, v_ref, gc_ref):
    x = x_ref[:]
    xn = x * jax.lax.rsqrt(jnp.mean(x * x, axis=-1, keepdims=True) + 1e-6)
    xn = xn * g_ref[:]
    q_ref[:] = _bdot(xn, wq_ref[:])
    k_ref[:] = _bdot(xn, wk_ref[:])
    v_ref[:] = _bdot(xn, wv_ref[:])
    gc_ref[:] = jax.nn.sigmoid(_bdot(xn, wc_ref[:]) + bc_ref[:])


def _compress_kernel(kr_ref, vr_ref, kpe_ref, vpe_ref, wck_ref, wcv_ref,
                     mem_ref, ckf_ref, cvf_ref):
    k8 = kr_ref[0]
    v8 = vr_ref[0]
    half = STRIDE * DH
    kpe = kpe_ref[0]
    vpe = vpe_ref[0]
    k8s = jnp.concatenate([k8[1:], k8[:1]], axis=0)
    v8s = jnp.concatenate([v8[1:], v8[:1]], axis=0)
    kbf = jnp.concatenate([k8 + kpe[:, :half], k8s + kpe[:, half:]], axis=1)
    vbf = jnp.concatenate([v8 + vpe[:, :half], v8s + vpe[:, half:]], axis=1)
    ck = _bdot(kbf, wck_ref[0])
    cv = _bdot(vbf, wcv_ref[0])
    ckf_ref[0] = jnp.concatenate([mem_ref[0, 0], ck[:NCB]], axis=0)
    cvf_ref[0] = jnp.concatenate([mem_ref[1, 0], cv[:NCB]], axis=0)


def _attn_kernel(q_ref, k_ref, v_ref, ckf_ref, cvf_ref,
                 co_ref, fo_ref, so_ref):
    i = pl.program_id(1)
    q = q_ref[0]
    ckf = ckf_ref[0]
    cvf = cvf_ref[0]
    t = i * TQ + jax.lax.broadcasted_iota(jnp.int32, (TQ, 1), 0)

    cs = _bdot(q, ckf, (((1,), (1,)), ((), ()))) * SCALE
    jc = jax.lax.broadcasted_iota(jnp.int32, (TQ, NCB + 1), 1)
    cmask = (jc == 0) | ((jc - 1) * STRIDE + BLK - 1 <= t)
    cs = jnp.where(cmask, cs, -1e30)
    cm = jnp.max(cs, axis=-1, keepdims=True)
    ce = jnp.exp(cs - cm)
    cp = ce / jnp.sum(ce, axis=-1, keepdims=True)
    co_ref[0] = _bdot(cp, cvf)

    cpr = jnp.concatenate([cp[:, 1:], cp[:, :1]], axis=1)
    impw = cp + cpr
    lane = jax.lax.broadcasted_iota(jnp.int32, (TQ, NCB + 1), 1)
    odd = (lane % 2) == 1
    impw = jnp.where(odd, impw, -1.0)
    impw = jnp.where(lane == NCB, cp[:, NCB:NCB + 1], impw)
    sels = []
    for _ in range(NSEL):
        m = jnp.max(impw, axis=-1, keepdims=True)
        idx = jnp.min(jnp.where(impw == m, lane, NCB + 1),
                      axis=-1, keepdims=True)
        sels.append(jax.lax.shift_right_logical(idx, 1))
        impw = jnp.where(lane == idx, -2.0, impw)

    s = _bdot(q, k_ref[0], (((1,), (1,)), ((), ()))) * SCALE
    jcol = jax.lax.broadcasted_iota(jnp.int32, (TQ, N), 1)
    causal = jcol <= t
    jb = jcol // SELBLK
    fmask = (jb == sels[0]) | (jb == sels[1]) | (jb == sels[2]) \
        | (jb == sels[3]) | (jb == t // SELBLK)
    fmask = fmask & causal
    fs = jnp.where(fmask, s, -1e30)
    fm = jnp.max(fs, axis=-1, keepdims=True)
    fe = jnp.exp(fs - fm)
    fp = fe / jnp.sum(fe, axis=-1, keepdims=True)
    fo_ref[0] = _bdot(fp, v_ref[0])

    smask = causal & (t - jcol < WIN)
    ws = jnp.where(smask, s, -1e30)
    wm = jnp.max(ws, axis=-1, keepdims=True)
    we = jnp.exp(ws - wm)
    wp = we / jnp.sum(we, axis=-1, keepdims=True)
    so_ref[0] = _bdot(wp, v_ref[0])


def _out_kernel(co_ref, fo_ref, so_ref, gc_ref, wo_ref, o_ref):
    gc = gc_ref[:]
    acc = jnp.zeros((TQ, D), F32)
    for h in range(H):
        comb = gc[:, h:h + 1] * co_ref[h] \
            + gc[:, H + h:H + h + 1] * fo_ref[h] \
            + gc[:, 2 * H + h:2 * H + h + 1] * so_ref[h]
        acc = acc + _bdot(comb, wo_ref[h])
    o_ref[:] = acc


def kernel(x, g_norm, Wq, Wk, Wv, k_pe, v_pe, Wck, Wcv, mem_kv,
           W_comb, b_comb, Wo):
    b, n, d = x.shape
    x2 = x.reshape(n, d)
    g2 = g_norm.reshape(1, d)
    b2 = b_comb.reshape(1, 3 * H)

    q, k, v, gc = pl.pallas_call(
        _proj_kernel,
        grid=(n // TQ,),
        in_specs=[
            pl.BlockSpec((TQ, d), lambda i: (i, 0)),
            pl.BlockSpec((1, d), lambda i: (0, 0)),
            pl.BlockSpec((d, H * DH), lambda i: (0, 0)),
            pl.BlockSpec((d, KVH * DH), lambda i: (0, 0)),
            pl.BlockSpec((d, KVH * DH), lambda i: (0, 0)),
            pl.BlockSpec((d, 3 * H), lambda i: (0, 0)),
            pl.BlockSpec((1, 3 * H), lambda i: (0, 0)),
        ],
        out_specs=[
            pl.BlockSpec((TQ, H * DH), lambda i: (i, 0)),
            pl.BlockSpec((TQ, KVH * DH), lambda i: (i, 0)),
            pl.BlockSpec((TQ, KVH * DH), lambda i: (i, 0)),
            pl.BlockSpec((TQ, 3 * H), lambda i: (i, 0)),
        ],
        out_shape=[
            jax.ShapeDtypeStruct((n, H * DH), F32),
            jax.ShapeDtypeStruct((n, KVH * DH), F32),
            jax.ShapeDtypeStruct((n, KVH * DH), F32),
            jax.ShapeDtypeStruct((n, 3 * H), F32),
        ],
    )(x2, g2, Wq, Wk, Wv, W_comb, b2)

    qh = q.reshape(n, H, DH).transpose(1, 0, 2)
    kh = k.reshape(n, KVH, DH).transpose(1, 0, 2)
    vh = v.reshape(n, KVH, DH).transpose(1, 0, 2)
    kr = kh.reshape(KVH, n // STRIDE, STRIDE * DH)
    vr = vh.reshape(KVH, n // STRIDE, STRIDE * DH)
    kpe2 = k_pe.reshape(KVH, 1, BLK * DH)
    vpe2 = v_pe.reshape(KVH, 1, BLK * DH)
    mem2 = mem_kv.reshape(2, KVH, 1, DH)

    ckf, cvf = pl.pallas_call(
        _compress_kernel,
        grid=(KVH,),
        in_specs=[
            pl.BlockSpec((1, n // STRIDE, STRIDE * DH), lambda h: (h, 0, 0)),
            pl.BlockSpec((1, n // STRIDE, STRIDE * DH), lambda h: (h, 0, 0)),
            pl.BlockSpec((1, 1, BLK * DH), lambda h: (h, 0, 0)),
            pl.BlockSpec((1, 1, BLK * DH), lambda h: (h, 0, 0)),
            pl.BlockSpec((1, BLK * DH, DH), lambda h: (h, 0, 0)),
            pl.BlockSpec((1, BLK * DH, DH), lambda h: (h, 0, 0)),
            pl.BlockSpec((2, 1, 1, DH), lambda h: (0, h, 0, 0)),
        ],
        out_specs=[
            pl.BlockSpec((1, NCB + 1, DH), lambda h: (h, 0, 0)),
            pl.BlockSpec((1, NCB + 1, DH), lambda h: (h, 0, 0)),
        ],
        out_shape=[
            jax.ShapeDtypeStruct((KVH, NCB + 1, DH), F32),
            jax.ShapeDtypeStruct((KVH, NCB + 1, DH), F32),
        ],
    )(kr, vr, kpe2, vpe2, Wck, Wcv, mem2)

    co, fo, so = pl.pallas_call(
        _attn_kernel,
        grid=(H, n // TQ),
        in_specs=[
            pl.BlockSpec((1, TQ, DH), lambda h, i: (h, i, 0)),
            pl.BlockSpec((1, n, DH), lambda h, i: (h, 0, 0)),
            pl.BlockSpec((1, n, DH), lambda h, i: (h, 0, 0)),
            pl.BlockSpec((1, NCB + 1, DH), lambda h, i: (h, 0, 0)),
            pl.BlockSpec((1, NCB + 1, DH), lambda h, i: (h, 0, 0)),
        ],
        out_specs=[
            pl.BlockSpec((1, TQ, DH), lambda h, i: (h, i, 0)),
            pl.BlockSpec((1, TQ, DH), lambda h, i: (h, i, 0)),
            pl.BlockSpec((1, TQ, DH), lambda h, i: (h, i, 0)),
        ],
        out_shape=[
            jax.ShapeDtypeStruct((H, n, DH), F32),
            jax.ShapeDtypeStruct((H, n, DH), F32),
            jax.ShapeDtypeStruct((H, n, DH), F32),
        ],
    )(qh, kh, vh, ckf, cvf)

    out = pl.pallas_call(
        _out_kernel,
        grid=(n // TQ,),
        in_specs=[
            pl.BlockSpec((H, TQ, DH), lambda i: (0, i, 0)),
            pl.BlockSpec((H, TQ, DH), lambda i: (0, i, 0)),
            pl.BlockSpec((H, TQ, DH), lambda i: (0, i, 0)),
            pl.BlockSpec((TQ, 3 * H), lambda i: (i, 0)),
            pl.BlockSpec((H, DH, d), lambda i: (0, 0, 0)),
        ],
        out_specs=pl.BlockSpec((TQ, d), lambda i: (i, 0)),
        out_shape=jax.ShapeDtypeStruct((n, d), F32),
    )(co, fo, so, gc, Wo.reshape(H, DH, d))

    return out.reshape(b, n, d)
